# baseline (speedup 1.0000x reference)
import functools

import jax
import jax.numpy as jnp
from jax.experimental import pallas as pl
from jax.experimental.pallas import tpu as pltpu

N_CELLS = 360
FEAT = 16
LAT = 64
BAND = 8
BATCH = 32
CIOS = 2844

NP = 384
SEG = 368
NSEG = 18
EH = NSEG * SEG
CB = 2


def _mm(x, w):
    return jax.lax.dot_general(x, w, (((1,), (0,)), ((), ())),
                               preferred_element_type=jnp.float32)


def _ln_rows(x, g, b):
    m = jnp.mean(x, axis=-1, keepdims=True)
    v = jnp.mean((x - m) ** 2, axis=-1, keepdims=True)
    return (x - m) * jax.lax.rsqrt(v + 1e-5) * g + b


def _gnn_kernel(obs_ref, act_ref, few_ref, neg_ref, neb_ref, aew_ref,
                outw_ref, outb_ref, m64_ref, v64_ref, out_ref,
                h_ref, agg_ref):
    mi = iter(range(64))
    vi = iter(range(64))

    def M():
        return m64_ref[next(mi)]

    def V():
        return v64_ref[next(vi)].reshape(1, LAT)

    cls = V(); fe_b = V(); ae_b = V(); na_g = V(); na_b = V()

    x = jnp.maximum(_mm(obs_ref[...].reshape(CB * N_CELLS, FEAT), few_ref[...])
                    .reshape(CB, N_CELLS, LAT) + fe_b, 0.0)
    m = jnp.mean(x, axis=(1, 2), keepdims=True)
    v = jnp.mean((x - m) ** 2, axis=(1, 2), keepdims=True)
    x = (x - m) * jax.lax.rsqrt(v + 1e-5) * neg_ref[...] + neb_ref[...]

    agg_ref[...] = jnp.zeros((CB, NP, LAT), jnp.float32)
    agg_ref[:, 0:1, :] = jnp.broadcast_to(cls, (CB, 1, LAT))
    agg_ref[:, 1:N_CELLS + 1, :] = x
    node = agg_ref[...]

    a = _ln_rows(_mm(act_ref[0], aew_ref[...]) + ae_b, na_g, na_b)

    row = jax.lax.broadcasted_iota(jnp.int32, (N_CELLS, 1), 0)

    efeat = None
    for i in range(2):
        w1s = M(); w1d = M()
        w1e = M() if i > 0 else None
        w2 = M(); w3 = M(); v1a = M(); v1b = M(); v2 = M(); v3 = M()
        eb1 = V(); eb2 = V(); eb3 = V(); vb1 = V(); vb2 = V(); vb3 = V()
        nng = V(); nnb = V(); eng = V(); enb = V()

        n2d = node.reshape(CB * NP, LAT)
        pp = _mm(n2d, w1s).reshape(CB, NP, LAT)
        qp = (_mm(n2d, w1d) + eb1).reshape(CB, NP, LAT)

        h_ref[:, 0:SEG, :] = pp[:, 0:1, :] + qp[:, 1:1 + SEG, :]
        h_ref[:, SEG:2 * SEG, :] = pp[:, 1:1 + SEG, :] + qp[:, 0:1, :]
        for d in range(1, BAND + 1):
            s = (1 + d) * SEG
            h_ref[:, s:s + SEG, :] = pp[:, d + 1:d + 1 + SEG, :] + qp[:, 1:1 + SEG, :]
        for d in range(1, BAND + 1):
            s = (9 + d) * SEG
            h_ref[:, s:s + SEG, :] = pp[:, 1:1 + SEG, :] + qp[:, d + 1:d + 1 + SEG, :]

        h1 = h_ref[...].reshape(CB * EH, LAT)
        if i > 0:
            h1 = h1 + _mm(efeat, w1e)
        h1 = jnp.maximum(h1, 0.0)
        h2 = jnp.maximum(_mm(h1, w2) + eb2, 0.0)
        ef2d = _mm(h2, w3) + eb3
        ef = ef2d.reshape(CB, EH, LAT)

        agg_ref[...] = jnp.zeros((CB, NP, LAT), jnp.float32)
        agg_ref[:, 1:N_CELLS + 1, :] += ef[:, 0:N_CELLS, :]
        agg_ref[:, 0:1, :] += jnp.sum(ef[:, SEG:SEG + N_CELLS, :], axis=1,
                                      keepdims=True)
        for d in range(1, BAND + 1):
            s = (1 + d) * SEG
            msk = jnp.where(row < N_CELLS - d, 1.0, 0.0)
            agg_ref[:, 1:N_CELLS + 1, :] += ef[:, s:s + N_CELLS, :] * msk
        for d in range(1, BAND + 1):
            s = (9 + d) * SEG
            agg_ref[:, d + 1:d + 1 + N_CELLS, :] += ef[:, s:s + N_CELLS, :]
        agg2d = agg_ref[...].reshape(CB * NP, LAT)

        g1 = jnp.maximum(_mm(n2d, v1a) + _mm(agg2d, v1b) + vb1, 0.0)
        g2 = jnp.maximum(_mm(g1, v2) + vb2, 0.0)
        g3 = _mm(g2, v3) + vb3
        node = _ln_rows(g3, nng, nnb).reshape(CB, NP, LAT)

        if i == 0:
            efeat = _ln_rows(ef2d, eng, enb)
        else:
            efeat = _ln_rows(ef[:, SEG:2 * SEG, :].reshape(CB * SEG, LAT),
                             eng, enb)

    w1s = M(); w1d = M(); w1e = M(); w2 = M(); w3 = M()
    v1a = M(); v1b = M(); v2 = M(); v3 = M()
    eb1 = V(); eb2 = V(); eb3 = V(); vb1 = V(); vb2 = V(); vb3 = V()
    nng = V(); nnb = V()

    node0 = node[:, 0, :]
    pp = _mm(node.reshape(CB * NP, LAT), w1s).reshape(CB, NP, LAT)
    q0 = (_mm(node0, w1d) + eb1).reshape(CB, 1, LAT)
    h1 = (pp[:, 1:1 + SEG, :] + q0).reshape(CB * SEG, LAT) + _mm(efeat, w1e)
    h1 = jnp.maximum(h1, 0.0)
    h2 = jnp.maximum(_mm(h1, w2) + eb2, 0.0)
    ef = (_mm(h2, w3) + eb3).reshape(CB, SEG, LAT)
    agg0 = jnp.sum(ef[:, 0:N_CELLS, :], axis=1)

    g1 = jnp.maximum(_mm(node0, v1a) + _mm(agg0, v1b) + vb1, 0.0)
    g2 = jnp.maximum(_mm(g1, v2) + vb2, 0.0)
    g3 = _mm(g2, v3) + vb3
    node0 = _ln_rows(g3, nng, nnb)

    out_ref[0] = (_mm(node0, outw_ref[0:LAT, :])
                  + _mm(a, outw_ref[LAT:2 * LAT, :]) + outb_ref[...])


@jax.jit
def _run(obs3, actions, few, neg, neb, aew, outw, outb, m64, v64):
    grid = (BATCH // CB,)
    zi = lambda i: tuple(0 for _ in range(9))
    specs = [
        pl.BlockSpec((CB, N_CELLS, FEAT), lambda i: (i, 0, 0)),
        pl.BlockSpec((1, CB, CIOS), lambda i: (i, 0, 0)),
        pl.BlockSpec(few.shape, lambda i: (0, 0)),
        pl.BlockSpec(neg.shape, lambda i: (0, 0)),
        pl.BlockSpec(neb.shape, lambda i: (0, 0)),
        pl.BlockSpec(aew.shape, lambda i: (0, 0)),
        pl.BlockSpec(outw.shape, lambda i: (0, 0)),
        pl.BlockSpec(outb.shape, lambda i: (0, 0)),
        pl.BlockSpec(m64.shape, lambda i: (0, 0, 0)),
        pl.BlockSpec(v64.shape, lambda i: (0, 0)),
    ]
    return pl.pallas_call(
        _gnn_kernel,
        grid=grid,
        in_specs=specs,
        out_specs=pl.BlockSpec((1, CB, CIOS), lambda i: (i, 0, 0)),
        out_shape=jax.ShapeDtypeStruct((BATCH // CB, CB, CIOS), jnp.float32),
        scratch_shapes=[pltpu.VMEM((CB, EH, LAT), jnp.float32),
                        pltpu.VMEM((CB, NP, LAT), jnp.float32)],
        compiler_params=pltpu.CompilerParams(
            dimension_semantics=("parallel",)),
    )(obs3, actions.reshape(BATCH // CB, CB, CIOS), few, neg, neb, aew,
      outw, outb, m64, v64).reshape(BATCH, CIOS)


def kernel(observations, actions, params, edge_index):
    p = params
    obs3 = observations.reshape(BATCH, N_CELLS, FEAT)

    mats = []
    vecs = [p['cls'].reshape(LAT), p['fe_b'], p['ae_b'], p['na_g'], p['na_b']]
    for i in range(3):
        ew = p['emlp%d_w' % i]
        eb = p['emlp%d_b' % i]
        nw = p['nmlp%d_w' % i]
        nb = p['nmlp%d_b' % i]
        mats += [ew[0][0:LAT], ew[0][LAT:2 * LAT]]
        if i > 0:
            mats += [ew[0][2 * LAT:3 * LAT]]
        mats += [ew[1], ew[2], nw[0][0:LAT], nw[0][LAT:2 * LAT], nw[1], nw[2]]
        vecs += [eb[0], eb[1], eb[2], nb[0], nb[1], nb[2],
                 p['nn%d_g' % i], p['nn%d_b' % i]]
        if i < 2:
            vecs += [p['en%d_g' % i], p['en%d_b' % i]]
    m64 = jnp.stack(mats)
    v64 = jnp.stack(vecs)

    out = _run(obs3, actions, p['fe_w'], p['ne_g'], p['ne_b'], p['ae_w'],
               p['out_w'], p['out_b'].reshape(1, CIOS), m64, v64)
    return out[:, :, None]


# speedup vs baseline: 1.0637x; 1.0637x over previous
import functools

import jax
import jax.numpy as jnp
from jax.experimental import pallas as pl
from jax.experimental.pallas import tpu as pltpu

N_CELLS = 360
FEAT = 16
LAT = 64
BAND = 8
BATCH = 32
CIOS = 2844

NP = 384
SEG = 368
NSEG = 18
EH = NSEG * SEG
CB = 4


def _mm(x, w):
    return jax.lax.dot_general(x, w, (((1,), (0,)), ((), ())),
                               preferred_element_type=jnp.float32)


def _ln_rows(x, g, b):
    m = jnp.mean(x, axis=-1, keepdims=True)
    v = jnp.mean((x - m) ** 2, axis=-1, keepdims=True)
    return (x - m) * jax.lax.rsqrt(v + 1e-5) * g + b


def _gnn_kernel(obs_ref, act_ref, few_ref, neg_ref, neb_ref, aew_ref,
                outw_ref, outb_ref, m64_ref, v64_ref, out_ref,
                h_ref, agg_ref):
    mi = iter(range(64))
    vi = iter(range(64))

    def M():
        return m64_ref[next(mi)]

    def V():
        return v64_ref[next(vi)].reshape(1, LAT)

    cls = V(); fe_b = V(); ae_b = V(); na_g = V(); na_b = V()

    x = jnp.maximum(_mm(obs_ref[...].reshape(CB * N_CELLS, FEAT), few_ref[...])
                    .reshape(CB, N_CELLS, LAT) + fe_b, 0.0)
    m = jnp.mean(x, axis=(1, 2), keepdims=True)
    v = jnp.mean((x - m) ** 2, axis=(1, 2), keepdims=True)
    x = (x - m) * jax.lax.rsqrt(v + 1e-5) * neg_ref[...] + neb_ref[...]

    agg_ref[...] = jnp.zeros((CB, NP, LAT), jnp.float32)
    agg_ref[:, 0:1, :] = jnp.broadcast_to(cls, (CB, 1, LAT))
    agg_ref[:, 1:N_CELLS + 1, :] = x
    node = agg_ref[...]

    a = _ln_rows(_mm(act_ref[0], aew_ref[...]) + ae_b, na_g, na_b)

    row = jax.lax.broadcasted_iota(jnp.int32, (N_CELLS, 1), 0)

    efeat = None
    for i in range(2):
        w1s = M(); w1d = M()
        w1e = M() if i > 0 else None
        w2 = M(); w3 = M(); v1a = M(); v1b = M(); v2 = M(); v3 = M()
        eb1 = V(); eb2 = V(); eb3 = V(); vb1 = V(); vb2 = V(); vb3 = V()
        nng = V(); nnb = V(); eng = V(); enb = V()

        n2d = node.reshape(CB * NP, LAT)
        pp = _mm(n2d, w1s).reshape(CB, NP, LAT)
        qp = (_mm(n2d, w1d) + eb1).reshape(CB, NP, LAT)

        h_ref[:, 0:SEG, :] = pp[:, 0:1, :] + qp[:, 1:1 + SEG, :]
        h_ref[:, SEG:2 * SEG, :] = pp[:, 1:1 + SEG, :] + qp[:, 0:1, :]
        for d in range(1, BAND + 1):
            s = (1 + d) * SEG
            h_ref[:, s:s + SEG, :] = pp[:, d + 1:d + 1 + SEG, :] + qp[:, 1:1 + SEG, :]
        for d in range(1, BAND + 1):
            s = (9 + d) * SEG
            h_ref[:, s:s + SEG, :] = pp[:, 1:1 + SEG, :] + qp[:, d + 1:d + 1 + SEG, :]

        h1 = h_ref[...].reshape(CB * EH, LAT)
        if i > 0:
            h1 = h1 + _mm(efeat, w1e)
        h1 = jnp.maximum(h1, 0.0)
        h2 = jnp.maximum(_mm(h1, w2) + eb2, 0.0)
        ef2d = _mm(h2, w3) + eb3
        ef = ef2d.reshape(CB, EH, LAT)

        agg_ref[...] = jnp.zeros((CB, NP, LAT), jnp.float32)
        agg_ref[:, 1:N_CELLS + 1, :] += ef[:, 0:N_CELLS, :]
        agg_ref[:, 0:1, :] += jnp.sum(ef[:, SEG:SEG + N_CELLS, :], axis=1,
                                      keepdims=True)
        for d in range(1, BAND + 1):
            s = (1 + d) * SEG
            msk = jnp.where(row < N_CELLS - d, 1.0, 0.0)
            agg_ref[:, 1:N_CELLS + 1, :] += ef[:, s:s + N_CELLS, :] * msk
        for d in range(1, BAND + 1):
            s = (9 + d) * SEG
            agg_ref[:, d + 1:d + 1 + N_CELLS, :] += ef[:, s:s + N_CELLS, :]
        agg2d = agg_ref[...].reshape(CB * NP, LAT)

        g1 = jnp.maximum(_mm(n2d, v1a) + _mm(agg2d, v1b) + vb1, 0.0)
        g2 = jnp.maximum(_mm(g1, v2) + vb2, 0.0)
        g3 = _mm(g2, v3) + vb3
        node = _ln_rows(g3, nng, nnb).reshape(CB, NP, LAT)

        if i == 0:
            efeat = _ln_rows(ef2d, eng, enb)
        else:
            efeat = _ln_rows(ef[:, SEG:2 * SEG, :].reshape(CB * SEG, LAT),
                             eng, enb)

    w1s = M(); w1d = M(); w1e = M(); w2 = M(); w3 = M()
    v1a = M(); v1b = M(); v2 = M(); v3 = M()
    eb1 = V(); eb2 = V(); eb3 = V(); vb1 = V(); vb2 = V(); vb3 = V()
    nng = V(); nnb = V()

    node0 = node[:, 0, :]
    pp = _mm(node.reshape(CB * NP, LAT), w1s).reshape(CB, NP, LAT)
    q0 = (_mm(node0, w1d) + eb1).reshape(CB, 1, LAT)
    h1 = (pp[:, 1:1 + SEG, :] + q0).reshape(CB * SEG, LAT) + _mm(efeat, w1e)
    h1 = jnp.maximum(h1, 0.0)
    h2 = jnp.maximum(_mm(h1, w2) + eb2, 0.0)
    ef = (_mm(h2, w3) + eb3).reshape(CB, SEG, LAT)
    agg0 = jnp.sum(ef[:, 0:N_CELLS, :], axis=1)

    g1 = jnp.maximum(_mm(node0, v1a) + _mm(agg0, v1b) + vb1, 0.0)
    g2 = jnp.maximum(_mm(g1, v2) + vb2, 0.0)
    g3 = _mm(g2, v3) + vb3
    node0 = _ln_rows(g3, nng, nnb)

    out_ref[0] = (_mm(node0, outw_ref[0:LAT, :])
                  + _mm(a, outw_ref[LAT:2 * LAT, :]) + outb_ref[...])


@jax.jit
def _run(obs3, actions, few, neg, neb, aew, outw, outb, m64, v64):
    grid = (BATCH // CB,)
    zi = lambda i: tuple(0 for _ in range(9))
    specs = [
        pl.BlockSpec((CB, N_CELLS, FEAT), lambda i: (i, 0, 0)),
        pl.BlockSpec((1, CB, CIOS), lambda i: (i, 0, 0)),
        pl.BlockSpec(few.shape, lambda i: (0, 0)),
        pl.BlockSpec(neg.shape, lambda i: (0, 0)),
        pl.BlockSpec(neb.shape, lambda i: (0, 0)),
        pl.BlockSpec(aew.shape, lambda i: (0, 0)),
        pl.BlockSpec(outw.shape, lambda i: (0, 0)),
        pl.BlockSpec(outb.shape, lambda i: (0, 0)),
        pl.BlockSpec(m64.shape, lambda i: (0, 0, 0)),
        pl.BlockSpec(v64.shape, lambda i: (0, 0)),
    ]
    return pl.pallas_call(
        _gnn_kernel,
        grid=grid,
        in_specs=specs,
        out_specs=pl.BlockSpec((1, CB, CIOS), lambda i: (i, 0, 0)),
        out_shape=jax.ShapeDtypeStruct((BATCH // CB, CB, CIOS), jnp.float32),
        scratch_shapes=[pltpu.VMEM((CB, EH, LAT), jnp.float32),
                        pltpu.VMEM((CB, NP, LAT), jnp.float32)],
        compiler_params=pltpu.CompilerParams(
            dimension_semantics=("parallel",)),
    )(obs3, actions.reshape(BATCH // CB, CB, CIOS), few, neg, neb, aew,
      outw, outb, m64, v64).reshape(BATCH, CIOS)


def kernel(observations, actions, params, edge_index):
    p = params
    obs3 = observations.reshape(BATCH, N_CELLS, FEAT)

    mats = []
    vecs = [p['cls'].reshape(LAT), p['fe_b'], p['ae_b'], p['na_g'], p['na_b']]
    for i in range(3):
        ew = p['emlp%d_w' % i]
        eb = p['emlp%d_b' % i]
        nw = p['nmlp%d_w' % i]
        nb = p['nmlp%d_b' % i]
        mats += [ew[0][0:LAT], ew[0][LAT:2 * LAT]]
        if i > 0:
            mats += [ew[0][2 * LAT:3 * LAT]]
        mats += [ew[1], ew[2], nw[0][0:LAT], nw[0][LAT:2 * LAT], nw[1], nw[2]]
        vecs += [eb[0], eb[1], eb[2], nb[0], nb[1], nb[2],
                 p['nn%d_g' % i], p['nn%d_b' % i]]
        if i < 2:
            vecs += [p['en%d_g' % i], p['en%d_b' % i]]
    m64 = jnp.stack(mats)
    v64 = jnp.stack(vecs)

    out = _run(obs3, actions, p['fe_w'], p['ne_g'], p['ne_b'], p['ae_w'],
               p['out_w'], p['out_b'].reshape(1, CIOS), m64, v64)
    return out[:, :, None]


# speedup vs baseline: 1.1749x; 1.1045x over previous
import jax
import jax.numpy as jnp
from jax.experimental import pallas as pl
from jax.experimental.pallas import tpu as pltpu

N_CELLS = 360
FEAT = 16
LAT = 64
BAND = 8
BATCH = 32
CIOS = 2844

NP = 384
SEG = 368
NSEG = 18
EH = NSEG * SEG
CB = 4


def _mm(x, w):
    return jax.lax.dot_general(x, w, (((1,), (0,)), ((), ())),
                               preferred_element_type=jnp.float32)


def _ln_rows(x, g, b):
    m = jnp.mean(x, axis=-1, keepdims=True)
    v = jnp.mean((x - m) ** 2, axis=-1, keepdims=True)
    return (x - m) * jax.lax.rsqrt(v + 1e-5) * g + b


def _gnn_kernel(*refs):
    it = iter(refs[:-3])
    out_ref, h_ref, agg_ref = refs[-3:]

    obs_ref = next(it); act_ref = next(it)
    few = next(it)[...]; neg = next(it)[...]; neb = next(it)[...]
    aew_ref = next(it); outw_ref = next(it); outb = next(it)[...]
    cls = next(it)[...]; feb = next(it)[...]; aeb = next(it)[...]
    nag = next(it)[...]; nab = next(it)[...]
    stk = []
    for i in range(3):
        s = {k: next(it) for k in
             ('ew1', 'ew2', 'ew3', 'eb1', 'eb2', 'eb3',
              'nw1', 'nw2', 'nw3', 'nb1', 'nb2', 'nb3', 'nng', 'nnb')}
        if i < 2:
            s['eng'] = next(it); s['enb'] = next(it)
        stk.append(s)

    x = jnp.maximum(_mm(obs_ref[...].reshape(CB * N_CELLS, FEAT), few)
                    .reshape(CB, N_CELLS, LAT) + feb, 0.0)
    m = jnp.mean(x, axis=(1, 2), keepdims=True)
    v = jnp.mean((x - m) ** 2, axis=(1, 2), keepdims=True)
    x = (x - m) * jax.lax.rsqrt(v + 1e-5) * neg + neb

    agg_ref[:, 0:1, :] = jnp.broadcast_to(cls, (CB, 1, LAT))
    agg_ref[:, 1:N_CELLS + 1, :] = x
    agg_ref[:, N_CELLS + 1:, :] = jnp.zeros((CB, NP - N_CELLS - 1, LAT),
                                            jnp.float32)
    node = agg_ref[...]

    a = _ln_rows(_mm(act_ref[0], aew_ref[...]) + aeb, nag, nab)

    vi = jax.lax.broadcasted_iota(jnp.int32, (NP, 1), 0).astype(jnp.float32)
    deg = jnp.where(
        vi == 0.0, float(N_CELLS),
        jnp.where(vi <= float(N_CELLS),
                  1.0 + jnp.clip(float(N_CELLS) - vi, 0.0, float(BAND))
                  + jnp.clip(vi - 1.0, 0.0, float(BAND)),
                  0.0))

    def build_h(node):
        n2d = node.reshape(CB * NP, LAT)
        pp = _mm(n2d, w1s).reshape(CB, NP, LAT)
        qp = (_mm(n2d, w1d) + s['eb1'][...]).reshape(CB, NP, LAT)
        h_ref[:, 0:SEG, :] = pp[:, 0:1, :] + qp[:, 1:1 + SEG, :]
        h_ref[:, SEG:2 * SEG, :] = pp[:, 1:1 + SEG, :] + qp[:, 0:1, :]
        for d in range(1, BAND + 1):
            o = (1 + d) * SEG
            h_ref[:, o:o + SEG, :] = (pp[:, d + 1:d + 1 + SEG, :]
                                      + qp[:, 1:1 + SEG, :])
        for d in range(1, BAND + 1):
            o = (9 + d) * SEG
            h_ref[:, o:o + SEG, :] = (pp[:, 1:1 + SEG, :]
                                      + qp[:, d + 1:d + 1 + SEG, :])
        return h_ref[...].reshape(CB * EH, LAT)

    def scatter(ef):
        s_band = ef[:, 0:N_CELLS, :]
        for d in range(1, BAND + 1):
            o = (1 + d) * SEG
            s_band = s_band + ef[:, o:o + N_CELLS, :]
        agg_ref[:, 0:1, :] = jnp.sum(ef[:, SEG:SEG + N_CELLS, :], axis=1,
                                     keepdims=True)
        agg_ref[:, 1:N_CELLS + 1, :] = s_band
        agg_ref[:, N_CELLS + 1:, :] = jnp.zeros((CB, NP - N_CELLS - 1, LAT),
                                                jnp.float32)
        for d in range(1, BAND + 1):
            o = (1 + d) * SEG + N_CELLS - d
            agg_ref[:, N_CELLS + 1 - d:N_CELLS + 1, :] += -ef[:, o:o + d, :]
        for d in range(1, BAND + 1):
            o = (9 + d) * SEG
            agg_ref[:, d + 1:d + 1 + N_CELLS, :] += ef[:, o:o + N_CELLS, :]
        return agg_ref[...]

    efeat = None
    for i in range(2):
        s = stk[i]
        ew1 = s['ew1']
        w1s = ew1[0:LAT, :]; w1d = ew1[LAT:2 * LAT, :]
        h1 = build_h(node)
        if i > 0:
            h1 = h1 + _mm(efeat, ew1[2 * LAT:3 * LAT, :])
        h1 = jnp.maximum(h1, 0.0)
        h2 = jnp.maximum(_mm(h1, s['ew2'][...]) + s['eb2'][...], 0.0)
        if i == 0:
            ef2d = _mm(h2, s['ew3'][...]) + s['eb3'][...]
            agg = scatter(ef2d.reshape(CB, EH, LAT))
            efeat = _ln_rows(ef2d, s['eng'][...], s['enb'][...])
        else:
            aggh = scatter(h2.reshape(CB, EH, LAT))
            agg = (_mm(aggh.reshape(CB * NP, LAT), s['ew3'][...])
                   .reshape(CB, NP, LAT) + (deg * s['eb3'][...])[None])
            h2hub = h2.reshape(CB, EH, LAT)[:, SEG:2 * SEG, :]
            efhub = (_mm(h2hub.reshape(CB * SEG, LAT), s['ew3'][...])
                     + s['eb3'][...])
            efeat = _ln_rows(efhub, s['eng'][...], s['enb'][...])

        n2d = node.reshape(CB * NP, LAT)
        g1 = jnp.maximum(_mm(n2d, s['nw1'][0:LAT, :])
                         + _mm(agg.reshape(CB * NP, LAT),
                               s['nw1'][LAT:2 * LAT, :])
                         + s['nb1'][...], 0.0)
        g2 = jnp.maximum(_mm(g1, s['nw2'][...]) + s['nb2'][...], 0.0)
        g3 = _mm(g2, s['nw3'][...]) + s['nb3'][...]
        node = _ln_rows(g3, s['nng'][...], s['nnb'][...]).reshape(CB, NP, LAT)

    s = stk[2]
    ew1 = s['ew1']
    node0 = node[:, 0, :]
    pp = _mm(node.reshape(CB * NP, LAT), ew1[0:LAT, :]).reshape(CB, NP, LAT)
    q0 = (_mm(node0, ew1[LAT:2 * LAT, :]) + s['eb1'][...]).reshape(CB, 1, LAT)
    h1 = ((pp[:, 1:1 + SEG, :] + q0).reshape(CB * SEG, LAT)
          + _mm(efeat, ew1[2 * LAT:3 * LAT, :]))
    h1 = jnp.maximum(h1, 0.0)
    h2 = jnp.maximum(_mm(h1, s['ew2'][...]) + s['eb2'][...], 0.0)
    h2sum = jnp.sum(h2.reshape(CB, SEG, LAT)[:, 0:N_CELLS, :], axis=1)
    agg0 = _mm(h2sum, s['ew3'][...]) + float(N_CELLS) * s['eb3'][...]

    g1 = jnp.maximum(_mm(node0, s['nw1'][0:LAT, :])
                     + _mm(agg0, s['nw1'][LAT:2 * LAT, :]) + s['nb1'][...], 0.0)
    g2 = jnp.maximum(_mm(g1, s['nw2'][...]) + s['nb2'][...], 0.0)
    g3 = _mm(g2, s['nw3'][...]) + s['nb3'][...]
    node0 = _ln_rows(g3, s['nng'][...], s['nnb'][...])

    out_ref[0] = (_mm(node0, outw_ref[0:LAT, :])
                  + _mm(a, outw_ref[LAT:2 * LAT, :]) + outb)


@jax.jit
def _run(obs3, act3, *weights):
    grid = (BATCH // CB,)

    def _const_spec(arr):
        nd = arr.ndim
        return pl.BlockSpec(arr.shape, lambda i, _n=nd: (0,) * _n)

    specs = ([pl.BlockSpec((CB, N_CELLS, FEAT), lambda i: (i, 0, 0)),
              pl.BlockSpec((1, CB, CIOS), lambda i: (i, 0, 0))]
             + [_const_spec(w) for w in weights])
    return pl.pallas_call(
        _gnn_kernel,
        grid=grid,
        in_specs=specs,
        out_specs=pl.BlockSpec((1, CB, CIOS), lambda i: (i, 0, 0)),
        out_shape=jax.ShapeDtypeStruct((BATCH // CB, CB, CIOS), jnp.float32),
        scratch_shapes=[pltpu.VMEM((CB, EH, LAT), jnp.float32),
                        pltpu.VMEM((CB, NP, LAT), jnp.float32)],
        compiler_params=pltpu.CompilerParams(
            dimension_semantics=("arbitrary",)),
    )(obs3, act3, *weights).reshape(BATCH, CIOS)


def kernel(observations, actions, params, edge_index):
    p = params
    r = lambda v: v.reshape(1, LAT)
    weights = [p['fe_w'], p['ne_g'], p['ne_b'], p['ae_w'], p['out_w'],
               p['out_b'].reshape(1, CIOS), p['cls'], r(p['fe_b']),
               r(p['ae_b']), r(p['na_g']), r(p['na_b'])]
    for i in range(3):
        ew = p['emlp%d_w' % i]; eb = p['emlp%d_b' % i]
        nw = p['nmlp%d_w' % i]; nb = p['nmlp%d_b' % i]
        weights += [ew[0], ew[1], ew[2], r(eb[0]), r(eb[1]), r(eb[2]),
                    nw[0], nw[1], nw[2], r(nb[0]), r(nb[1]), r(nb[2]),
                    r(p['nn%d_g' % i]), r(p['nn%d_b' % i])]
        if i < 2:
            weights += [r(p['en%d_g' % i]), r(p['en%d_b' % i])]
    out = _run(observations.reshape(BATCH, N_CELLS, FEAT),
               actions.reshape(BATCH // CB, CB, CIOS), *weights)
    return out[:, :, None]
